# baseline (speedup 1.0000x reference)
import functools

import jax
import jax.numpy as jnp
from jax.experimental import pallas as pl
from jax.experimental.pallas import tpu as pltpu

_BR = 256
_T = _BR // 8
_G = 4
_NV = 64
_ABS = 0x7FFFFFFF


def _tile(x_ref, t, c):
    return x_ref[t * 8:(t + 1) * 8, c * 128:(c + 1) * 128]


def _absf(v):
    return jax.lax.bitcast_convert_type(
        jax.lax.bitcast_convert_type(v, jnp.int32) & _ABS, jnp.float32)


def _combined_group(x_ref, ts, iota):
    accs = {t: jnp.zeros((8, 128), jnp.float32) for t in ts}
    mins = {t: jnp.full((8, 128), jnp.inf, jnp.float32) for t in ts}
    prev = {t: _tile(x_ref, t, 0) for t in ts}
    for c in range(_NV):
        for t in ts:
            xt = _tile(x_ref, t, c)
            rolled = pltpu.roll(xt, 1, 1)
            shifted = jnp.where(iota == 0.0, prev[t], rolled)
            accs[t] = accs[t] + jnp.abs(xt - shifted)
            mins[t] = jnp.minimum(mins[t], _absf(xt))
            prev[t] = rolled
    return accs, mins


def _masked_tile(x_ref, t, iota):
    rows = slice(t * 8, (t + 1) * 8)

    def cbody(c, cnt):
        xt = x_ref[rows, pl.ds(c * 128, 128)]
        return cnt + jnp.where(xt != 0.0, 1.0, 0.0)

    cnt = jax.lax.fori_loop(0, _NV, cbody,
                            jnp.zeros((8, 128), jnp.float32))
    real_len = jnp.sum(cnt, axis=1, keepdims=True)

    def body(c, carry):
        acc, prev = carry
        xt = x_ref[rows, pl.ds(c * 128, 128)]
        rolled = pltpu.roll(xt, 1, 1)
        shifted = jnp.where(iota == 0.0, prev, rolled)
        d = jnp.abs(xt - shifted)
        thresh = real_len - (c * 128).astype(jnp.float32)
        acc = acc + jnp.where(iota < thresh, d, 0.0)
        return acc, rolled

    init = (jnp.zeros((8, 128), jnp.float32), x_ref[rows, 0:128])
    acc, _ = jax.lax.fori_loop(0, _NV, body, init)
    rowsum = jnp.sum(acc, axis=1, keepdims=True)
    return rowsum / real_len


def _body(x_ref, out_ref, *, seq):
    i = pl.program_id(0)
    iota = jax.lax.broadcasted_iota(
        jnp.int32, (8, 128), 1).astype(jnp.float32)

    accs, mins = [], []
    for g in range(0, _T, _G):
        a, m = _combined_group(x_ref, list(range(g, g + _G)), iota)
        accs.extend(a[t] for t in range(g, g + _G))
        mins.extend(m[t] for t in range(g, g + _G))

    mn = mins[0]
    for m in mins[1:]:
        mn = jnp.minimum(mn, m)
    no_zero = jnp.min(mn) > 0.0

    row_id0 = (jax.lax.broadcasted_iota(jnp.int32, (8, 128), 0)
               + i * _BR).astype(jnp.float32)

    def fast():
        a0 = jnp.where(row_id0 >= 1.0, accs[0], 0.0)
        tot = a0
        for t in range(1, _T):
            tot = tot + accs[t]
        tot = jnp.sum(tot, axis=1, keepdims=True) * (1.0 / float(seq))
        return jnp.sum(tot, axis=0, keepdims=True)

    def slow():
        tot = None
        for t in range(_T):
            per_row = _masked_tile(x_ref, t, iota)
            row_id = (jax.lax.broadcasted_iota(jnp.int32, (8, 1), 0)
                      + (i * _BR + t * 8)).astype(jnp.float32)
            per_row = jnp.where(row_id >= 1.0, per_row, 0.0)
            tot = per_row if tot is None else tot + per_row
        return jnp.sum(tot, axis=0, keepdims=True)

    tot = jax.lax.cond(no_zero, fast, slow)
    out_ref[...] = jnp.broadcast_to(tot[None], (1, 1, 128))


def _consecutive_loss(x):
    bsz, seq = x.shape
    nb = bsz // _BR
    partials = pl.pallas_call(
        functools.partial(_body, seq=seq),
        grid=(nb,),
        in_specs=[pl.BlockSpec((_BR, seq), lambda i: (i, 0))],
        out_specs=pl.BlockSpec((1, 1, 128), lambda i: (i, 0, 0)),
        out_shape=jax.ShapeDtypeStruct((nb, 1, 128), jnp.float32),
        compiler_params=pltpu.CompilerParams(
            dimension_semantics=("parallel",),
        ),
    )(x)
    return jnp.sum(partials[:, 0, 0]) / bsz


def kernel(x):
    return _consecutive_loss(x)


# speedup vs baseline: 1.0639x; 1.0639x over previous
import functools

import jax
import jax.numpy as jnp
from jax.experimental import pallas as pl
from jax.experimental.pallas import tpu as pltpu

_BR = 512
_T = _BR // 8
_G = 4
_NV = 64
_ABS = 0x7FFFFFFF


def _tile(x_ref, t, c):
    return x_ref[t * 8:(t + 1) * 8, c * 128:(c + 1) * 128]


def _absf(v):
    return jax.lax.bitcast_convert_type(
        jax.lax.bitcast_convert_type(v, jnp.int32) & _ABS, jnp.float32)


def _combined_group(x_ref, ts, iota):
    accs = {t: jnp.zeros((8, 128), jnp.float32) for t in ts}
    mins = {t: jnp.full((8, 128), jnp.inf, jnp.float32) for t in ts}
    prev = {t: _tile(x_ref, t, 0) for t in ts}
    for c in range(_NV):
        for t in ts:
            xt = _tile(x_ref, t, c)
            rolled = pltpu.roll(xt, 1, 1)
            shifted = jnp.where(iota == 0.0, prev[t], rolled)
            accs[t] = accs[t] + jnp.abs(xt - shifted)
            mins[t] = jnp.minimum(mins[t], _absf(xt))
            prev[t] = rolled
    return accs, mins


def _masked_tile(x_ref, t, iota):
    rows = slice(t * 8, (t + 1) * 8)

    def cbody(c, cnt):
        xt = x_ref[rows, pl.ds(c * 128, 128)]
        return cnt + jnp.where(xt != 0.0, 1.0, 0.0)

    cnt = jax.lax.fori_loop(0, _NV, cbody,
                            jnp.zeros((8, 128), jnp.float32))
    real_len = jnp.sum(cnt, axis=1, keepdims=True)

    def body(c, carry):
        acc, prev = carry
        xt = x_ref[rows, pl.ds(c * 128, 128)]
        rolled = pltpu.roll(xt, 1, 1)
        shifted = jnp.where(iota == 0.0, prev, rolled)
        d = jnp.abs(xt - shifted)
        thresh = real_len - (c * 128).astype(jnp.float32)
        acc = acc + jnp.where(iota < thresh, d, 0.0)
        return acc, rolled

    init = (jnp.zeros((8, 128), jnp.float32), x_ref[rows, 0:128])
    acc, _ = jax.lax.fori_loop(0, _NV, body, init)
    rowsum = jnp.sum(acc, axis=1, keepdims=True)
    return rowsum / real_len


def _body(x_ref, x2_ref, out_ref, *, seq):
    i = pl.program_id(0)
    refs = (x_ref, x2_ref)
    iota = jax.lax.broadcasted_iota(
        jnp.int32, (8, 128), 1).astype(jnp.float32)

    accs, mins = [], []
    half = _T // 2
    for g in range(0, _T, _G):
        ref = refs[g // half]
        a, m = _combined_group(ref, list(range(g % half, g % half + _G)), iota)
        accs.extend(a[t] for t in range(g % half, g % half + _G))
        mins.extend(m[t] for t in range(g % half, g % half + _G))

    mn = mins[0]
    for m in mins[1:]:
        mn = jnp.minimum(mn, m)
    no_zero = jnp.min(mn) > 0.0

    row_id0 = (jax.lax.broadcasted_iota(jnp.int32, (8, 128), 0)
               + i * _BR).astype(jnp.float32)

    def fast():
        a0 = jnp.where(row_id0 >= 1.0, accs[0], 0.0)
        tot = a0
        for t in range(1, _T):
            tot = tot + accs[t]
        tot = jnp.sum(tot, axis=1, keepdims=True) * (1.0 / float(seq))
        return jnp.sum(tot, axis=0, keepdims=True)

    def slow():
        tot = None
        half = _T // 2
        for t in range(_T):
            per_row = _masked_tile(refs[t // half], t % half, iota)
            row_id = (jax.lax.broadcasted_iota(jnp.int32, (8, 1), 0)
                      + (i * _BR + t * 8)).astype(jnp.float32)
            per_row = jnp.where(row_id >= 1.0, per_row, 0.0)
            tot = per_row if tot is None else tot + per_row
        return jnp.sum(tot, axis=0, keepdims=True)

    tot = jax.lax.cond(no_zero, fast, slow)
    out_ref[...] = jnp.broadcast_to(tot[None], (1, 1, 128))


def _consecutive_loss(x):
    bsz, seq = x.shape
    nb = bsz // _BR
    partials = pl.pallas_call(
        functools.partial(_body, seq=seq),
        grid=(nb,),
        in_specs=[pl.BlockSpec((_BR // 2, seq), lambda i: (2 * i, 0)),
                  pl.BlockSpec((_BR // 2, seq), lambda i: (2 * i + 1, 0))],
        out_specs=pl.BlockSpec((1, 1, 128), lambda i: (i, 0, 0)),
        out_shape=jax.ShapeDtypeStruct((nb, 1, 128), jnp.float32),
        compiler_params=pltpu.CompilerParams(
            dimension_semantics=("parallel",),
        ),
    )(x, x)
    return jnp.sum(partials[:, 0, 0]) / bsz


def kernel(x):
    return _consecutive_loss(x)
